# baseline (speedup 1.0000x reference)
import jax
import jax.numpy as jnp
import numpy as np
from jax.experimental import pallas as pl

_NUM_SAMPLES = 128
_NEAR = 0.1
_FAR = 100.0
_STEP = (_FAR - _NEAR) / (_NUM_SAMPLES - 1)
_BN = 4096


def _plane_weights():
    s = _NUM_SAMPLES
    depth = (_NEAR + np.arange(s) * np.float32(_STEP)).astype(np.float32)
    w = np.zeros((6, 6 * s), dtype=np.float32)
    for c in range(3):
        w[c, c * s : (c + 1) * s] = 1.0
        w[3 + c, c * s : (c + 1) * s] = depth
        w[3 + c, (3 + c) * s : (4 + c) * s] = 1.0
    return jnp.asarray(w)


def _tc_body(o_ref, d_ref, w_ref, pos_ref, view_ref, dep_ref, del_ref):
    ot = o_ref[:]
    dt = d_ref[:]
    dn = dt / (jnp.sqrt(jnp.sum(dt * dt, axis=0, keepdims=True)) + 1e-8)
    a = jnp.concatenate([ot, dn], axis=0)
    s = _NUM_SAMPLES
    for c in range(3):
        big = jax.lax.dot_general(
            a, w_ref[:, c * s : (c + 1) * s], (((0,), (0,)), ((), ())),
            preferred_element_type=jnp.float32,
            precision=jax.lax.Precision.HIGHEST,
        )
        pos_ref[c] = big
        view_ref[c] = jax.lax.dot_general(
            a, w_ref[:, (3 + c) * s : (4 + c) * s], (((0,), (0,)), ((), ())),
            preferred_element_type=jnp.float32,
            precision=jax.lax.Precision.HIGHEST,
        )
    lane = jax.lax.broadcasted_iota(jnp.int32, (1, s), 1)
    depth_row = _NEAR + lane.astype(jnp.float32) * _STEP
    dep_ref[:] = jnp.broadcast_to(depth_row, (_BN, s))
    del_ref[:] = jnp.full((_BN, s), _STEP, dtype=jnp.float32)


@jax.jit
def kernel(origins, directions):
    n, _ = origins.shape
    s = _NUM_SAMPLES
    grid = (n // _BN,)
    pos3, view3, depths, deltas = pl.pallas_call(
        _tc_body,
        grid=grid,
        in_specs=[
            pl.BlockSpec((3, _BN), lambda i: (0, i)),
            pl.BlockSpec((3, _BN), lambda i: (0, i)),
            pl.BlockSpec((6, 6 * s), lambda i: (0, 0)),
        ],
        out_specs=[
            pl.BlockSpec((3, _BN, s), lambda i: (0, i, 0)),
            pl.BlockSpec((3, _BN, s), lambda i: (0, i, 0)),
            pl.BlockSpec((_BN, s), lambda i: (i, 0)),
            pl.BlockSpec((_BN, s), lambda i: (i, 0)),
        ],
        out_shape=[
            jax.ShapeDtypeStruct((3, n, s), jnp.float32),
            jax.ShapeDtypeStruct((3, n, s), jnp.float32),
            jax.ShapeDtypeStruct((n, s), jnp.float32),
            jax.ShapeDtypeStruct((n, s), jnp.float32),
        ],
    )(origins.T, directions.T, _plane_weights())
    positions = pos3.transpose(1, 2, 0)
    view_directions = view3.transpose(1, 2, 0)
    return positions, view_directions, depths, deltas


# speedup vs baseline: 1.8274x; 1.8274x over previous
import jax
import jax.numpy as jnp
from jax.experimental import pallas as pl
from jax.experimental.pallas import tpu as pltpu

_NUM_SAMPLES = 128
_NEAR = 0.1
_FAR = 100.0
_STEP = (_FAR - _NEAR) / (_NUM_SAMPLES - 1)
_BN = 4096


def _tc_body(o_ref, d_ref, pos_ref, view_ref, dep_ref, del_ref):
    o = jnp.transpose(o_ref[:])
    d = jnp.transpose(d_ref[:])
    dn = d / (jnp.sqrt(jnp.sum(d * d, axis=1, keepdims=True)) + 1e-8)
    lane = jax.lax.broadcasted_iota(jnp.int32, (1, _NUM_SAMPLES), 1)
    depth_row = _NEAR + lane.astype(jnp.float32) * _STEP
    depth_b = jnp.broadcast_to(depth_row, (_BN, _NUM_SAMPLES))
    for c in range(3):
        oc = jnp.broadcast_to(o[:, c : c + 1], (_BN, _NUM_SAMPLES))
        vc = jnp.broadcast_to(dn[:, c : c + 1], (_BN, _NUM_SAMPLES))
        view_ref[c] = vc
        pos_ref[c] = oc + vc * depth_b
    dep_ref[:] = depth_b
    del_ref[:] = jnp.full((_BN, _NUM_SAMPLES), _STEP, dtype=jnp.float32)


@jax.jit
def kernel(origins, directions):
    n, _ = origins.shape
    s = _NUM_SAMPLES
    grid = (n // _BN,)
    pos3, view3, depths, deltas = pl.pallas_call(
        _tc_body,
        grid=grid,
        in_specs=[
            pl.BlockSpec((3, _BN), lambda i: (0, i)),
            pl.BlockSpec((3, _BN), lambda i: (0, i)),
        ],
        out_specs=[
            pl.BlockSpec((3, _BN, s), lambda i: (0, i, 0)),
            pl.BlockSpec((3, _BN, s), lambda i: (0, i, 0)),
            pl.BlockSpec((_BN, s), lambda i: (i, 0)),
            pl.BlockSpec((_BN, s), lambda i: (i, 0)),
        ],
        out_shape=[
            jax.ShapeDtypeStruct((3, n, s), jnp.float32),
            jax.ShapeDtypeStruct((3, n, s), jnp.float32),
            jax.ShapeDtypeStruct((n, s), jnp.float32),
            jax.ShapeDtypeStruct((n, s), jnp.float32),
        ],
    )(origins.T, directions.T)
    positions = pos3.transpose(1, 2, 0)
    view_directions = view3.transpose(1, 2, 0)
    return positions, view_directions, depths, deltas


# speedup vs baseline: 2.0630x; 1.1289x over previous
import jax
import jax.numpy as jnp
import numpy as np
from jax.experimental import pallas as pl

_NUM_SAMPLES = 128
_NEAR = 0.1
_FAR = 100.0
_STEP = (_FAR - _NEAR) / (_NUM_SAMPLES - 1)
_BN = 4096


def _plane_weights():
    s = _NUM_SAMPLES
    depth = (_NEAR + np.arange(s) * np.float32(_STEP)).astype(np.float32)
    w = np.zeros((6, 6 * s), dtype=np.float32)
    for c in range(3):
        w[c, c * s : (c + 1) * s] = 1.0
        w[3 + c, c * s : (c + 1) * s] = depth
        w[3 + c, (3 + c) * s : (4 + c) * s] = 1.0
    return jnp.asarray(w)


def _tc_body(o_ref, d_ref, w_ref, pos_ref, view_ref, dep_ref, del_ref):
    ot = o_ref[:]
    dt = d_ref[:]
    dn = dt / (jnp.sqrt(jnp.sum(dt * dt, axis=0, keepdims=True)) + 1e-8)
    a = jnp.concatenate([ot, dn], axis=0)
    s = _NUM_SAMPLES
    for c in range(3):
        pos_ref[c] = jax.lax.dot_general(
            a, w_ref[:, c * s : (c + 1) * s], (((0,), (0,)), ((), ())),
            preferred_element_type=jnp.float32,
        )
        view_ref[c] = jax.lax.dot_general(
            a, w_ref[:, (3 + c) * s : (4 + c) * s], (((0,), (0,)), ((), ())),
            preferred_element_type=jnp.float32,
        )
    lane = jax.lax.broadcasted_iota(jnp.int32, (1, s), 1)
    depth_row = _NEAR + lane.astype(jnp.float32) * _STEP
    dep_ref[:] = jnp.broadcast_to(depth_row, (_BN, s))
    del_ref[:] = jnp.full((_BN, s), _STEP, dtype=jnp.float32)


@jax.jit
def kernel(origins, directions):
    n, _ = origins.shape
    s = _NUM_SAMPLES
    grid = (n // _BN,)
    pos3, view3, depths, deltas = pl.pallas_call(
        _tc_body,
        grid=grid,
        in_specs=[
            pl.BlockSpec((3, _BN), lambda i: (0, i)),
            pl.BlockSpec((3, _BN), lambda i: (0, i)),
            pl.BlockSpec((6, 6 * s), lambda i: (0, 0)),
        ],
        out_specs=[
            pl.BlockSpec((3, _BN, s), lambda i: (0, i, 0)),
            pl.BlockSpec((3, _BN, s), lambda i: (0, i, 0)),
            pl.BlockSpec((_BN, s), lambda i: (i, 0)),
            pl.BlockSpec((_BN, s), lambda i: (i, 0)),
        ],
        out_shape=[
            jax.ShapeDtypeStruct((3, n, s), jnp.float32),
            jax.ShapeDtypeStruct((3, n, s), jnp.float32),
            jax.ShapeDtypeStruct((n, s), jnp.float32),
            jax.ShapeDtypeStruct((n, s), jnp.float32),
        ],
    )(origins.T, directions.T, _plane_weights())
    positions = pos3.transpose(1, 2, 0)
    view_directions = view3.transpose(1, 2, 0)
    return positions, view_directions, depths, deltas
